# Initial kernel scaffold; baseline (speedup 1.0000x reference)
import jax
import jax.numpy as jnp
from jax.experimental import pallas as pl


def kernel(hidden_states, st_mask, edges_src, edges_tgt, edges_type, edges_pos, max_token, max_sentence, sentence_start, max_paragraph):
    raise NotImplementedError("write your pallas kernel here")


# speedup vs baseline: 3.1227x; 3.1227x over previous
import dataclasses
import functools

import jax
import jax.numpy as jnp
from jax import lax
from jax.experimental import pallas as pl
from jax.experimental.pallas import tpu as pltpu
from jax.experimental.pallas import tpu_sc as plsc

TOK = 40960
SENT = 2048
PARA = 512
N_NODES = TOK + SENT + PARA + 1
PARA_START = TOK + SENT
PARA_END = PARA_START + PARA
STRIDE = 80
T2S = 1
P2D = 11
H = 256
HV = H // 16

NC, NS = 2, 16
PH = 6
OWN = 256
PH_ROWS = NS * OWN
HALF = PH * PH_ROWS
NPAD = NC * HALF
SLICE = 2048
SLICES = 8
PRCH = 512
XCAP = SLICES * SLICE + PRCH
FCAP = 1088
FLUSH_HI = 960


def _make_pass(E, etype, redirect):
  EPT = E // NS
  assert EPT == SLICE * SLICES
  mesh = plsc.VectorSubcoreMesh(core_axis_name="c", subcore_axis_name="s")
  cp = pltpu.CompilerParams()
  if "needs_layout_passes" in pltpu.CompilerParams.__dataclass_fields__:
    cp = dataclasses.replace(cp, needs_layout_passes=False)

  @functools.partial(
      pl.kernel,
      out_type=(
          jax.ShapeDtypeStruct((NPAD, H), jnp.float32),
          jax.ShapeDtypeStruct((NC, NS, SLICES, SLICE), jnp.int32),
          jax.ShapeDtypeStruct((NC, NS, SLICES, SLICE), jnp.int32),
          jax.ShapeDtypeStruct((NC, NS, XCAP), jnp.int32),
          jax.ShapeDtypeStruct((NC, NS, XCAP), jnp.int32),
          jax.ShapeDtypeStruct((NC, NS * 16), jnp.int32),
      ),
      mesh=mesh,
      compiler_params=cp,
      scratch_types=[
          pltpu.VMEM((SLICE,), jnp.int32),
          pltpu.VMEM((SLICE,), jnp.int32),
          pltpu.VMEM((SLICE,), jnp.int32),
          pltpu.VMEM((SLICE + 16,), jnp.int32),
          pltpu.VMEM((SLICE + 16,), jnp.int32),
          pltpu.VMEM((PRCH,), jnp.int32),
          pltpu.VMEM((PRCH,), jnp.int32),
          pltpu.VMEM((PRCH + 32,), jnp.int32),
          pltpu.VMEM((PRCH + 32,), jnp.int32),
          pltpu.VMEM((FCAP,), jnp.int32),
          pltpu.VMEM((FCAP,), jnp.int32),
          pltpu.VMEM((16, H), jnp.float32),
          pltpu.VMEM((16, H), jnp.float32),
          pltpu.VMEM((OWN * H,), jnp.float32),
          pltpu.VMEM((OWN * 16,), jnp.float32),
          pltpu.VMEM((16,), jnp.int32),
          pltpu.VMEM((NS * 16,), jnp.int32),
      ],
  )
  def pass_kernel(gh, esrc, etgt, etyp, out, hsrc, htgt, xsrc, xtgt, xcnt,
                  r_typ, r_src, r_tgt, tc_src, tc_tgt, pr_src, pr_tgt,
                  wsrc, wtgt, fsrc, ftgt, gstage, ghst, acc, cnt,
                  cpub, crd):
    c = lax.axis_index("c")
    s = lax.axis_index("s")
    zero16f = jnp.zeros((16,), jnp.float32)
    one16f = jnp.ones((16,), jnp.float32)
    i16 = lax.iota(jnp.int32, 16)

    counts = []
    for sl in range(SLICES):
      base = s * EPT + sl * SLICE
      pltpu.sync_copy(etyp.at[pl.ds(base, SLICE)], r_typ)
      pltpu.sync_copy(esrc.at[pl.ds(base, SLICE)], r_src)
      pltpu.sync_copy(etgt.at[pl.ds(base, SLICE)], r_tgt)

      def vec_body(i, n):
        t = r_typ[pl.ds(i, 16)]
        sv = r_src[pl.ds(i, 16)]
        gv = r_tgt[pl.ds(i, 16)]
        if redirect:
          is_para = (sv >= PARA_START) & (sv < PARA_END)
          sv = jnp.where(is_para, (sv - PARA_START) * STRIDE, sv)
        m = t == etype
        plsc.store_compressed(tc_src.at[pl.ds(n, 16)], sv, mask=m)
        plsc.store_compressed(tc_tgt.at[pl.ds(n, 16)], gv, mask=m)
        return n + jnp.sum(m.astype(jnp.int32))

      n_sl = pl.loop(0, SLICE, step=16, init_carry=jnp.int32(0))(vec_body)
      counts.append(n_sl)
      pltpu.sync_copy(tc_src.at[pl.ds(0, SLICE)], hsrc.at[c, s, sl])
      pltpu.sync_copy(tc_tgt.at[pl.ds(0, SLICE)], htgt.at[c, s, sl])

    def gflush(nf):
      ftgt[pl.ds(nf, 16)] = jnp.full((16,), -1, jnp.int32)
      fsrc[pl.ds(nf, 16)] = jnp.zeros((16,), jnp.int32)

      @pl.loop(0, (nf + 15) // 16)
      def _(b):
        off = b * 16
        pltpu.sync_copy(gh.at[fsrc.at[pl.ds(off, 16)]], gstage)
        lv = ftgt[pl.ds(off, 16)]
        for k in range(16):
          le = lv[k]

          @pl.when(le >= 0)
          def _():
            rb = le * H
            cvec = cnt[pl.ds(le * 16, 16)]
            mz = jnp.where(cvec > 0.0, one16f, zero16f)
            for h in range(HV):
              acc[pl.ds(rb + h * 16, 16)] = (
                  acc[pl.ds(rb + h * 16, 16)] * mz
                  + gstage[k, pl.ds(h * 16, 16)])
            cnt[pl.ds(le * 16, 16)] = cvec + one16f

      return jnp.int32(0)

    @pl.loop(0, PH)
    def _(p):
      plo = c * HALF + p * PH_ROWS
      my_lo = plo + s * OWN

      @pl.loop(0, OWN * 16, step=16)
      def _(i):
        cnt[pl.ds(i, 16)] = zero16f

      def spill(st):
        fp, nf = st
        pltpu.sync_copy(wsrc.at[pl.ds(0, PRCH)],
                        xsrc.at[c, s, pl.ds(fp * PRCH, PRCH)])
        pltpu.sync_copy(wtgt.at[pl.ds(0, PRCH)],
                        xtgt.at[c, s, pl.ds(fp * PRCH, PRCH)])
        wsrc[pl.ds(0, 16)] = wsrc[pl.ds(PRCH, 16)]
        wtgt[pl.ds(0, 16)] = wtgt[pl.ds(PRCH, 16)]
        return (fp + jnp.int32(1), nf - jnp.int32(PRCH))

      st = (jnp.int32(0), jnp.int32(0))
      for sl in range(SLICES):
        n_sl = counts[sl]

        def chunk_body(ci, st):
          off = ci * PRCH
          pltpu.sync_copy(hsrc.at[c, s, sl, pl.ds(off, PRCH)], pr_src)
          pltpu.sync_copy(htgt.at[c, s, sl, pl.ds(off, PRCH)], pr_tgt)

          def vec_body(i, st):
            fp, nf = st
            sv = pr_src[pl.ds(i, 16)]
            gv = pr_tgt[pl.ds(i, 16)]
            rel = gv - plo
            m = (rel >= 0) & (rel < PH_ROWS) & ((off + i + i16) < n_sl)
            plsc.store_compressed(wsrc.at[pl.ds(nf, 16)], sv, mask=m)
            plsc.store_compressed(wtgt.at[pl.ds(nf, 16)], gv, mask=m)
            nf = nf + jnp.sum(m.astype(jnp.int32))
            return lax.cond(nf >= PRCH, spill, lambda x: x, (fp, nf))

          return pl.loop(0, PRCH, step=16, init_carry=st)(vec_body)

        st = pl.loop(0, (n_sl + PRCH - 1) // PRCH, init_carry=st)(chunk_body)

      fp, nf = st

      @pl.when(nf > 0)
      def _():
        pltpu.sync_copy(wsrc.at[pl.ds(0, PRCH)],
                        xsrc.at[c, s, pl.ds(fp * PRCH, PRCH)])
        pltpu.sync_copy(wtgt.at[pl.ds(0, PRCH)],
                        xtgt.at[c, s, pl.ds(fp * PRCH, PRCH)])

      cpub[pl.ds(0, 16)] = jnp.full((16,), fp * PRCH + nf, jnp.int32)
      pltpu.sync_copy(cpub, xcnt.at[c, pl.ds(s * 16, 16)])
      plsc.subcore_barrier()

      pltpu.sync_copy(xcnt.at[c], crd)

      def writer_body(t, nf):
        cnt_t = crd[pl.ds(t * 16, 16)][0]

        def chunk_body(ci, nf):
          off = ci * PRCH
          pltpu.sync_copy(xsrc.at[c, t, pl.ds(off, PRCH)], pr_src)
          pltpu.sync_copy(xtgt.at[c, t, pl.ds(off, PRCH)], pr_tgt)

          def vec_body(i, nf):
            sv = pr_src[pl.ds(i, 16)]
            gv = pr_tgt[pl.ds(i, 16)]
            local = gv - my_lo
            m = (local >= 0) & (local < OWN) & ((off + i + i16) < cnt_t)
            plsc.store_compressed(fsrc.at[pl.ds(nf, 16)], sv, mask=m)
            plsc.store_compressed(ftgt.at[pl.ds(nf, 16)], local, mask=m)
            nf = nf + jnp.sum(m.astype(jnp.int32))
            return lax.cond(nf >= FLUSH_HI, gflush, lambda x: x, nf)

          return pl.loop(0, PRCH, step=16, init_carry=nf)(vec_body)

        return pl.loop(0, (cnt_t + PRCH - 1) // PRCH,
                       init_carry=nf)(chunk_body)

      nf = pl.loop(0, NS, init_carry=jnp.int32(0))(writer_body)
      nf = lax.cond(nf > 0, gflush, lambda x: x, nf)
      plsc.subcore_barrier()

      @pl.loop(0, OWN, step=16)
      def _(k):
        row0 = my_lo + k
        if redirect:
          ridx = row0 + i16
          is_para = (ridx >= PARA_START) & (ridx < PARA_END)
          ridx = jnp.where(is_para, (ridx - PARA_START) * STRIDE, ridx)
          pltpu.sync_copy(gh.at[ridx], ghst)
        else:
          pltpu.sync_copy(gh.at[pl.ds(row0, 16)], ghst)
        for r in range(16):
          l = k + r
          cvec = cnt[pl.ds(l * 16, 16)]
          cval = cvec[0]

          @pl.when(cval > 0.0)
          def _():
            rcp = one16f / cvec
            for h in range(HV):
              sl2 = pl.ds(h * 16, 16)
              ghst[r, sl2] = (ghst[r, sl2]
                              + acc[pl.ds(l * H + h * 16, 16)] * rcp)

        pltpu.sync_copy(ghst, out.at[pl.ds(row0, 16)])

  return pass_kernel


def kernel(hidden_states, st_mask, edges_src, edges_tgt, edges_type, edges_pos,
           max_token, max_sentence, sentence_start, max_paragraph):
  E = edges_src.shape[0]
  pad = jnp.zeros((NPAD - TOK, H), jnp.float32)
  gh0 = jnp.concatenate([hidden_states.astype(jnp.float32), pad], axis=0)
  o1 = _make_pass(E, T2S, False)(gh0, edges_src, edges_tgt, edges_type)
  o2 = _make_pass(E, P2D, True)(o1[0], edges_src, edges_tgt, edges_type)
  return o2[0][:N_NODES].reshape(1, N_NODES, H)


# speedup vs baseline: 4.2535x; 1.3621x over previous
import dataclasses
import functools

import jax
import jax.numpy as jnp
from jax import lax
from jax.experimental import pallas as pl
from jax.experimental.pallas import tpu as pltpu
from jax.experimental.pallas import tpu_sc as plsc

TOK = 40960
SENT = 2048
PARA = 512
N_NODES = TOK + SENT + PARA + 1
PARA_START = TOK + SENT
PARA_END = PARA_START + PARA
STRIDE = 80
T2S = 1
P2D = 11
H = 256
HV = H // 16

NC, NS = 2, 16
PH = 6
OWN = 256
PH_ROWS = NS * OWN
HALF = PH * PH_ROWS
NPAD = NC * HALF
SLICE = 4096
SLICES = 4
PRCH = 512
XCAP = SLICES * SLICE + PRCH
FCAP = 1088
FLUSH_HI = 960


def _make_pass(E, etype, redirect):
  EPT = E // NS
  assert EPT == SLICE * SLICES
  mesh = plsc.VectorSubcoreMesh(core_axis_name="c", subcore_axis_name="s")
  cp = pltpu.CompilerParams()
  if "needs_layout_passes" in pltpu.CompilerParams.__dataclass_fields__:
    cp = dataclasses.replace(cp, needs_layout_passes=False)

  @functools.partial(
      pl.kernel,
      out_type=(
          jax.ShapeDtypeStruct((NPAD, H), jnp.float32),
          jax.ShapeDtypeStruct((NC, NS, SLICES, SLICE), jnp.int32),
          jax.ShapeDtypeStruct((NC, NS, SLICES, SLICE), jnp.int32),
          jax.ShapeDtypeStruct((NC, NS, XCAP), jnp.int32),
          jax.ShapeDtypeStruct((NC, NS, XCAP), jnp.int32),
          jax.ShapeDtypeStruct((NC, NS * 16), jnp.int32),
      ),
      mesh=mesh,
      compiler_params=cp,
      scratch_types=[
          pltpu.VMEM((SLICE,), jnp.int32),
          pltpu.VMEM((SLICE,), jnp.int32),
          pltpu.VMEM((SLICE,), jnp.int32),
          pltpu.VMEM((SLICE + 16,), jnp.int32),
          pltpu.VMEM((SLICE + 16,), jnp.int32),
          pltpu.VMEM((PRCH,), jnp.int32),
          pltpu.VMEM((PRCH,), jnp.int32),
          pltpu.VMEM((PRCH + 32,), jnp.int32),
          pltpu.VMEM((PRCH + 32,), jnp.int32),
          pltpu.VMEM((FCAP,), jnp.int32),
          pltpu.VMEM((FCAP,), jnp.int32),
          pltpu.VMEM((16, H), jnp.float32),
          pltpu.VMEM((64, H), jnp.float32),
          pltpu.VMEM((64,), jnp.int32),
          pltpu.VMEM((OWN * H,), jnp.float32),
          pltpu.VMEM((OWN * 16,), jnp.float32),
          pltpu.VMEM((16,), jnp.int32),
          pltpu.VMEM((NS * 16,), jnp.int32),
      ],
  )
  def pass_kernel(gh, esrc, etgt, etyp, out, hsrc, htgt, xsrc, xtgt, xcnt,
                  r_typ, r_src, r_tgt, tc_src, tc_tgt, pr_src, pr_tgt,
                  wsrc, wtgt, fsrc, ftgt, gstage, ghst, ridx_b, acc, cnt,
                  cpub, crd):
    c = lax.axis_index("c")
    s = lax.axis_index("s")
    zero16f = jnp.zeros((16,), jnp.float32)
    one16f = jnp.ones((16,), jnp.float32)
    i16 = lax.iota(jnp.int32, 16)

    def popcount(m):
      return plsc.all_reduce_population_count(m)[0]

    counts = []
    for sl in range(SLICES):
      base = s * EPT + sl * SLICE
      pltpu.sync_copy(etyp.at[pl.ds(base, SLICE)], r_typ)
      pltpu.sync_copy(esrc.at[pl.ds(base, SLICE)], r_src)
      pltpu.sync_copy(etgt.at[pl.ds(base, SLICE)], r_tgt)

      def vec_body(i, n):
        t = r_typ[pl.ds(i, 16)]
        sv = r_src[pl.ds(i, 16)]
        gv = r_tgt[pl.ds(i, 16)]
        if redirect:
          is_para = (sv >= PARA_START) & (sv < PARA_END)
          sv = jnp.where(is_para, (sv - PARA_START) * STRIDE, sv)
        m = t == etype
        plsc.store_compressed(tc_src.at[pl.ds(n, 16)], sv, mask=m)
        plsc.store_compressed(tc_tgt.at[pl.ds(n, 16)], gv, mask=m)
        return n + popcount(m)

      n_sl = pl.loop(0, SLICE, step=16, init_carry=jnp.int32(0))(vec_body)
      counts.append(n_sl)
      pltpu.sync_copy(tc_src.at[pl.ds(0, SLICE)], hsrc.at[c, s, sl])
      pltpu.sync_copy(tc_tgt.at[pl.ds(0, SLICE)], htgt.at[c, s, sl])

    def gflush(nf):
      ftgt[pl.ds(nf, 16)] = jnp.full((16,), -1, jnp.int32)
      fsrc[pl.ds(nf, 16)] = jnp.zeros((16,), jnp.int32)

      @pl.loop(0, (nf + 15) // 16)
      def _(b):
        off = b * 16
        pltpu.sync_copy(gh.at[fsrc.at[pl.ds(off, 16)]], gstage)
        lv = ftgt[pl.ds(off, 16)]
        for k in range(16):
          le = lv[k]

          @pl.when(le >= 0)
          def _():
            rb = le * H
            cvec = cnt[pl.ds(le * 16, 16)]
            mz = jnp.where(cvec > 0.0, one16f, zero16f)
            for h in range(HV):
              acc[pl.ds(rb + h * 16, 16)] = (
                  acc[pl.ds(rb + h * 16, 16)] * mz
                  + gstage[k, pl.ds(h * 16, 16)])
            cnt[pl.ds(le * 16, 16)] = cvec + one16f

      return jnp.int32(0)

    @pl.loop(0, PH)
    def _(p):
      plo = c * HALF + p * PH_ROWS
      my_lo = plo + s * OWN

      @pl.loop(0, OWN * 16, step=16)
      def _(i):
        cnt[pl.ds(i, 16)] = zero16f

      def spill(st):
        fp, nf = st
        pltpu.sync_copy(wsrc.at[pl.ds(0, PRCH)],
                        xsrc.at[c, s, pl.ds(fp * PRCH, PRCH)])
        pltpu.sync_copy(wtgt.at[pl.ds(0, PRCH)],
                        xtgt.at[c, s, pl.ds(fp * PRCH, PRCH)])
        wsrc[pl.ds(0, 16)] = wsrc[pl.ds(PRCH, 16)]
        wtgt[pl.ds(0, 16)] = wtgt[pl.ds(PRCH, 16)]
        return (fp + jnp.int32(1), nf - jnp.int32(PRCH))

      st = (jnp.int32(0), jnp.int32(0))
      for sl in range(SLICES):
        n_sl = counts[sl]

        def chunk_body(ci, st):
          off = ci * PRCH
          pltpu.sync_copy(hsrc.at[c, s, sl, pl.ds(off, PRCH)], pr_src)
          pltpu.sync_copy(htgt.at[c, s, sl, pl.ds(off, PRCH)], pr_tgt)

          def vec_body(i, st):
            fp, nf = st
            sv = pr_src[pl.ds(i, 16)]
            gv = pr_tgt[pl.ds(i, 16)]
            rel = gv - plo
            m = (rel >= 0) & (rel < PH_ROWS) & ((off + i + i16) < n_sl)
            plsc.store_compressed(wsrc.at[pl.ds(nf, 16)], sv, mask=m)
            plsc.store_compressed(wtgt.at[pl.ds(nf, 16)], gv, mask=m)
            nf = nf + popcount(m)
            return lax.cond(nf >= PRCH, spill, lambda x: x, (fp, nf))

          return pl.loop(0, PRCH, step=16, init_carry=st)(vec_body)

        st = pl.loop(0, (n_sl + PRCH - 1) // PRCH, init_carry=st)(chunk_body)

      fp, nf = st

      @pl.when(nf > 0)
      def _():
        pltpu.sync_copy(wsrc.at[pl.ds(0, PRCH)],
                        xsrc.at[c, s, pl.ds(fp * PRCH, PRCH)])
        pltpu.sync_copy(wtgt.at[pl.ds(0, PRCH)],
                        xtgt.at[c, s, pl.ds(fp * PRCH, PRCH)])

      cpub[pl.ds(0, 16)] = jnp.full((16,), fp * PRCH + nf, jnp.int32)
      pltpu.sync_copy(cpub, xcnt.at[c, pl.ds(s * 16, 16)])
      plsc.subcore_barrier()

      pltpu.sync_copy(xcnt.at[c], crd)

      def writer_body(t, nf):
        cnt_t = crd[pl.ds(t * 16, 16)][0]

        def chunk_body(ci, nf):
          off = ci * PRCH
          pltpu.sync_copy(xsrc.at[c, t, pl.ds(off, PRCH)], pr_src)
          pltpu.sync_copy(xtgt.at[c, t, pl.ds(off, PRCH)], pr_tgt)

          def vec_body(i, nf):
            sv = pr_src[pl.ds(i, 16)]
            gv = pr_tgt[pl.ds(i, 16)]
            local = gv - my_lo
            m = (local >= 0) & (local < OWN) & ((off + i + i16) < cnt_t)
            plsc.store_compressed(fsrc.at[pl.ds(nf, 16)], sv, mask=m)
            plsc.store_compressed(ftgt.at[pl.ds(nf, 16)], local, mask=m)
            nf = nf + popcount(m)
            return lax.cond(nf >= FLUSH_HI, gflush, lambda x: x, nf)

          return pl.loop(0, PRCH, step=16, init_carry=nf)(vec_body)

        return pl.loop(0, (cnt_t + PRCH - 1) // PRCH,
                       init_carry=nf)(chunk_body)

      nf = pl.loop(0, NS, init_carry=jnp.int32(0))(writer_body)
      nf = lax.cond(nf > 0, gflush, lambda x: x, nf)
      plsc.subcore_barrier()

      @pl.loop(0, OWN, step=64)
      def _(k):
        row0 = my_lo + k
        if redirect:
          @pl.loop(0, 64, step=16)
          def _(j):
            rv = row0 + j + i16
            is_para = (rv >= PARA_START) & (rv < PARA_END)
            rv = jnp.where(is_para, (rv - PARA_START) * STRIDE, rv)
            ridx_b[pl.ds(j, 16)] = rv

          pltpu.sync_copy(gh.at[ridx_b], ghst)
        else:
          pltpu.sync_copy(gh.at[pl.ds(row0, 64)], ghst)

        @pl.loop(0, 64)
        def _(r):
          l = k + r
          cvec = cnt[pl.ds(l * 16, 16)]
          cval = cvec[0]

          @pl.when(cval > 0.0)
          def _():
            rcp = one16f / cvec
            for h in range(HV):
              sl2 = pl.ds(h * 16, 16)
              ghst[r, sl2] = (ghst[r, sl2]
                              + acc[pl.ds(l * H + h * 16, 16)] * rcp)

        pltpu.sync_copy(ghst, out.at[pl.ds(row0, 64)])

  return pass_kernel


def kernel(hidden_states, st_mask, edges_src, edges_tgt, edges_type, edges_pos,
           max_token, max_sentence, sentence_start, max_paragraph):
  E = edges_src.shape[0]
  pad = jnp.zeros((NPAD - TOK, H), jnp.float32)
  gh0 = jnp.concatenate([hidden_states.astype(jnp.float32), pad], axis=0)
  o1 = _make_pass(E, T2S, False)(gh0, edges_src, edges_tgt, edges_type)
  o2 = _make_pass(E, P2D, True)(o1[0], edges_src, edges_tgt, edges_type)
  return o2[0][:N_NODES].reshape(1, N_NODES, H)


# speedup vs baseline: 6.8374x; 1.6075x over previous
import dataclasses
import functools

import jax
import jax.numpy as jnp
from jax import lax
from jax.experimental import pallas as pl
from jax.experimental.pallas import tpu as pltpu
from jax.experimental.pallas import tpu_sc as plsc

TOK = 40960
SENT = 2048
PARA = 512
N_NODES = TOK + SENT + PARA + 1
PARA_START = TOK + SENT
PARA_END = PARA_START + PARA
STRIDE = 80
T2S = 1
P2D = 11
H = 256
HV = H // 16

NC, NS = 2, 16
PH = 6
OWN = 256
PH_ROWS = NS * OWN
HALF = PH * PH_ROWS
NPAD = NC * HALF
SLICE = 4096
SLICES = 4
PAGE = 512
ZT = PAGE + 16
PGS = PAGE + ZT
WROW = PGS + 16
MAXPG = SLICES * SLICE // PAGE + 1
XCAP = MAXPG * PGS
FCAP = 1088
FLUSH_HI = 960
GB = 32
CH = 32


def _make_pass(E, etype, redirect):
  EPT = E // NS
  assert EPT == SLICE * SLICES
  mesh = plsc.VectorSubcoreMesh(core_axis_name="c", subcore_axis_name="s")
  cp = pltpu.CompilerParams()
  if "needs_layout_passes" in pltpu.CompilerParams.__dataclass_fields__:
    cp = dataclasses.replace(cp, needs_layout_passes=False)

  @functools.partial(
      pl.kernel,
      out_type=(
          jax.ShapeDtypeStruct((NPAD, H), jnp.float32),
          jax.ShapeDtypeStruct((NC * NS * PH * XCAP,), jnp.int32),
          jax.ShapeDtypeStruct((NC * PH * NS * 16,), jnp.int32),
      ),
      mesh=mesh,
      compiler_params=cp,
      scratch_types=[
          pltpu.VMEM((SLICE,), jnp.int32),
          pltpu.VMEM((SLICE + 16,), jnp.int32),
          pltpu.VMEM((SLICE + 16,), jnp.int32),
      ] + [pltpu.VMEM((PAGE + 32,), jnp.int32)] * (2 * PH) + [
          pltpu.VMEM((PGS,), jnp.int32),
          pltpu.VMEM((PGS,), jnp.int32),
          pltpu.VMEM((PGS,), jnp.int32),
          pltpu.VMEM((FCAP,), jnp.int32),
          pltpu.VMEM((FCAP,), jnp.int32),
          pltpu.VMEM((GB, H), jnp.float32),
          pltpu.VMEM((CH, H), jnp.float32),
          pltpu.VMEM((CH, H), jnp.float32),
          pltpu.VMEM((CH,), jnp.int32),
          pltpu.VMEM((CH,), jnp.int32),
          pltpu.VMEM((OWN * H,), jnp.float32),
          pltpu.VMEM((OWN * 16,), jnp.float32),
          pltpu.VMEM((16,), jnp.int32),
          pltpu.VMEM((NS * 16,), jnp.int32),
          pltpu.SemaphoreType.DMA,
          pltpu.SemaphoreType.DMA,
          pltpu.SemaphoreType.DMA,
          pltpu.SemaphoreType.DMA,
          pltpu.SemaphoreType.DMA,
          pltpu.SemaphoreType.DMA,
      ],
  )
  def pass_kernel(gh, esrc, etgt, etyp, out, xch, xcnt,
                  r_typ, r_src, r_tgt, *rest):
    wsrcs = rest[0:PH]
    wtgts = rest[PH:2 * PH]
    (prbA, prbB, prbC, fsrc, ftgt, gstage, ghA, ghB, ridxA, ridxB,
     acc, cnt, cpub, crd, semA, semB, rdA, rdB, wrA, wrB) = rest[2 * PH:]
    c = lax.axis_index("c")
    s = lax.axis_index("s")
    zero16f = jnp.zeros((16,), jnp.float32)
    one16f = jnp.ones((16,), jnp.float32)
    i16 = lax.iota(jnp.int32, 16)

    def popcount(m):
      return plsc.all_reduce_population_count(m)[0]

    def xoff(cc, tt, p, pg):
      return pl.multiple_of(((cc * NS + tt) * PH + p) * XCAP + pg * PGS, 8)

    pstate = [(jnp.int32(0), jnp.int32(0))] * PH
    for sl in range(SLICES):
      base = s * EPT + sl * SLICE
      pltpu.sync_copy(etyp.at[pl.ds(base, SLICE)], r_typ)
      pltpu.sync_copy(esrc.at[pl.ds(base, SLICE)], r_src.at[pl.ds(0, SLICE)])
      pltpu.sync_copy(etgt.at[pl.ds(base, SLICE)], r_tgt.at[pl.ds(0, SLICE)])

      def compact_body(i, n):
        t = r_typ[pl.ds(i, 16)]
        sv = r_src[pl.ds(i, 16)]
        gv = r_tgt[pl.ds(i, 16)]
        if redirect:
          is_para = (sv >= PARA_START) & (sv < PARA_END)
          sv = jnp.where(is_para, (sv - PARA_START) * STRIDE, sv)
        m = t == etype
        plsc.store_compressed(r_src.at[pl.ds(n, 16)], sv, mask=m)
        plsc.store_compressed(r_tgt.at[pl.ds(n, 16)], gv, mask=m)
        return n + popcount(m)

      n_sl = pl.loop(0, SLICE, step=16, init_carry=jnp.int32(0),
                     unroll=4)(compact_body)

      def spill_fn(p):
        def spill(st):
          fp, nf = st
          base2 = xoff(c, s, p, fp)
          pltpu.sync_copy(wsrcs[p].at[pl.ds(0, PAGE)],
                          xch.at[pl.ds(base2, PAGE)])
          pltpu.sync_copy(wtgts[p].at[pl.ds(0, PAGE)],
                          xch.at[pl.ds(base2 + ZT, PAGE)])
          wsrcs[p][pl.ds(0, 16)] = wsrcs[p][pl.ds(PAGE, 16)]
          wtgts[p][pl.ds(0, 16)] = wtgts[p][pl.ds(PAGE, 16)]
          return (fp + jnp.int32(1), nf - jnp.int32(PAGE))

        return spill

      def bucket_body(i, st):
        sv = r_src[pl.ds(i, 16)]
        gv = r_tgt[pl.ds(i, 16)]
        valid = (i + i16) < n_sl
        nst = []
        for p in range(PH):
          fp, nf = st[p]
          rel = gv - (c * HALF + p * PH_ROWS)
          m = (rel >= 0) & (rel < PH_ROWS) & valid
          plsc.store_compressed(wsrcs[p].at[pl.ds(nf, 16)], sv, mask=m)
          plsc.store_compressed(wtgts[p].at[pl.ds(nf, 16)], gv, mask=m)
          nf = nf + popcount(m)
          nst.append(lax.cond(nf >= PAGE, spill_fn(p), lambda x: x, (fp, nf)))
        return tuple(nst)

      nup = ((n_sl + 15) // 16) * 16
      pstate = pl.loop(0, nup, step=16,
                       init_carry=tuple(pstate))(bucket_body)
      pstate = list(pstate)

    for p in range(PH):
      fp, nf = pstate[p]

      @pl.when(nf > 0)
      def _():
        base2 = xoff(c, s, p, fp)
        pltpu.sync_copy(wsrcs[p].at[pl.ds(0, PAGE)],
                        xch.at[pl.ds(base2, PAGE)])
        pltpu.sync_copy(wtgts[p].at[pl.ds(0, PAGE)],
                        xch.at[pl.ds(base2 + ZT, PAGE)])

      cpub[pl.ds(0, 16)] = jnp.full((16,), fp * PAGE + nf, jnp.int32)
      pltpu.sync_copy(cpub, xcnt.at[pl.ds(pl.multiple_of((c * PH + p) * 256 + s * 16, 8), 16)])

    plsc.subcore_barrier()

    def gflush(nf):
      for j in range(GB // 16):
        ftgt[pl.ds(nf + j * 16, 16)] = jnp.full((16,), -1, jnp.int32)
        fsrc[pl.ds(nf + j * 16, 16)] = jnp.zeros((16,), jnp.int32)

      @pl.loop(0, (nf + GB - 1) // GB)
      def _(b):
        off = b * GB
        pltpu.sync_copy(gh.at[fsrc.at[pl.ds(off, GB)]], gstage)

        @pl.loop(0, GB)
        def _(e):
          le = ftgt[pl.ds(off + e, 16)][0]

          @pl.when(le >= 0)
          def _():
            rb = le * H
            cvec = cnt[pl.ds(le * 16, 16)]
            mz = jnp.where(cvec > 0.0, one16f, zero16f)
            for h in range(HV):
              acc[pl.ds(rb + h * 16, 16)] = (
                  acc[pl.ds(rb + h * 16, 16)] * mz
                  + gstage[e, pl.ds(h * 16, 16)])
            cnt[pl.ds(le * 16, 16)] = cvec + one16f

      return jnp.int32(0)

    @pl.loop(0, PH)
    def _(p):
      my_lo = c * HALF + p * PH_ROWS + s * OWN

      @pl.loop(0, OWN * 16, step=16)
      def _(i):
        cnt[pl.ds(i, 16)] = zero16f

      pltpu.sync_copy(xcnt.at[pl.ds(pl.multiple_of((c * PH + p) * 256, 8), 256)], crd)

      def scan_page(buf, off, ub, cnt_t, nf):
        def vec_body(i, nf):
          sv = buf[pl.ds(i, 16)]
          gv = buf[pl.ds(ZT + i, 16)]
          local = gv - my_lo
          m = (local >= 0) & (local < OWN) & ((off + i + i16) < cnt_t)
          plsc.store_compressed(fsrc.at[pl.ds(nf, 16)], sv, mask=m)
          plsc.store_compressed(ftgt.at[pl.ds(nf, 16)], local, mask=m)
          nf = nf + popcount(m)
          return lax.cond(nf >= FLUSH_HI, gflush, lambda x: x, nf)

        return pl.loop(0, ub, step=16, init_carry=nf)(vec_body)

      def do_writer(t, buf, sem, nf):
        pltpu.make_async_copy(xch.at[pl.ds(0, PGS)], buf, sem).wait()
        cnt_t = crd[pl.ds(t * 16, 16)][0]
        nf = scan_page(buf, 0, jnp.minimum(jnp.int32(PAGE), cnt_t),
                       cnt_t, nf)
        tn = jnp.minimum(t + 2, NS - 1)
        pltpu.async_copy(xch.at[pl.ds(xoff(c, tn, p, 0), PGS)], buf, sem)

        def extra(pg, nf):
          pltpu.sync_copy(xch.at[pl.ds(xoff(c, t, p, pg), PGS)], prbC)
          off = pg * PAGE
          return scan_page(prbC, off,
                           jnp.minimum(jnp.int32(PAGE), cnt_t - off),
                           cnt_t, nf)

        return pl.loop(1, (cnt_t + PAGE - 1) // PAGE, init_carry=nf)(extra)

      pltpu.async_copy(xch.at[pl.ds(xoff(c, 0, p, 0), PGS)], prbA, semA)
      pltpu.async_copy(xch.at[pl.ds(xoff(c, 1, p, 0), PGS)], prbB, semB)

      def pair_body(u, nf):
        nf = do_writer(2 * u, prbA, semA, nf)
        return do_writer(2 * u + 1, prbB, semB, nf)

      nf = pl.loop(0, NS // 2, init_carry=jnp.int32(0))(pair_body)
      pltpu.make_async_copy(xch.at[pl.ds(0, PGS)], prbA, semA).wait()
      pltpu.make_async_copy(xch.at[pl.ds(0, PGS)], prbB, semB).wait()
      nf = lax.cond(nf > 0, gflush, lambda x: x, nf)

      def d_rdstart(k, buf, idxb, sem):
        row0 = my_lo + k * CH
        if redirect:
          @pl.loop(0, CH, step=16)
          def _(j):
            rv = row0 + j + i16
            is_para = (rv >= PARA_START) & (rv < PARA_END)
            rv = jnp.where(is_para, (rv - PARA_START) * STRIDE, rv)
            idxb[pl.ds(j, 16)] = rv

          pltpu.async_copy(gh.at[idxb], buf, sem)
        else:
          pltpu.async_copy(gh.at[pl.ds(row0, CH)], buf, sem)

      def d_step(k, buf, idxb, rsem, wsem):
        pltpu.make_async_copy(gh.at[pl.ds(0, CH)], buf, rsem).wait()

        @pl.loop(0, CH)
        def _(r):
          l = k * CH + r
          cvec = cnt[pl.ds(l * 16, 16)]
          cval = cvec[0]

          @pl.when(cval > 0.0)
          def _():
            rcp = one16f / cvec
            for h in range(HV):
              sl2 = pl.ds(h * 16, 16)
              buf[r, sl2] = (buf[r, sl2]
                             + acc[pl.ds(l * H + h * 16, 16)] * rcp)

        pltpu.async_copy(buf, out.at[pl.ds(my_lo + k * CH, CH)], wsem)

      NCH = OWN // CH
      d_rdstart(0, ghA, ridxA, rdA)

      @pl.loop(0, NCH // 2)
      def _(u):
        k0 = 2 * u

        @pl.when(u > 0)
        def _():
          pltpu.make_async_copy(ghB, out.at[pl.ds(0, CH)], wrB).wait()

        d_rdstart(k0 + 1, ghB, ridxB, rdB)
        d_step(k0, ghA, ridxA, rdA, wrA)
        pltpu.make_async_copy(ghA, out.at[pl.ds(0, CH)], wrA).wait()

        @pl.when(u < NCH // 2 - 1)
        def _():
          d_rdstart(k0 + 2, ghA, ridxA, rdA)

        d_step(k0 + 1, ghB, ridxB, rdB, wrB)

      pltpu.make_async_copy(ghB, out.at[pl.ds(0, CH)], wrB).wait()

  return pass_kernel


def kernel(hidden_states, st_mask, edges_src, edges_tgt, edges_type, edges_pos,
           max_token, max_sentence, sentence_start, max_paragraph):
  E = edges_src.shape[0]
  pad = jnp.zeros((NPAD - TOK, H), jnp.float32)
  gh0 = jnp.concatenate([hidden_states.astype(jnp.float32), pad], axis=0)
  o1 = _make_pass(E, T2S, False)(gh0, edges_src, edges_tgt, edges_type)
  o2 = _make_pass(E, P2D, True)(o1[0], edges_src, edges_tgt, edges_type)
  return o2[0][:N_NODES].reshape(1, N_NODES, H)
